# Initial kernel scaffold; baseline (speedup 1.0000x reference)
import jax
import jax.numpy as jnp
from jax.experimental import pallas as pl


def kernel(sparse_fea, dense_fea, stk_coor, dts_tconv_w, dts_tconv_b, dts_tbn_g, dts_tbn_b, dts_mlp_w, dts_mlp_b, dts_bn_g, dts_bn_b, std_mlp_w, std_mlp_b, std_bn_g, std_bn_b, sp_conv_w, sp_conv_b, sp_bn_g, sp_bn_b, dn_conv_w, dn_conv_b, dn_bn_g, dn_bn_b, dn_down_w, dn_down_b, dn_dbn_g, dn_dbn_b):
    raise NotImplementedError("write your pallas kernel here")


# speedup vs baseline: 2.0721x; 2.0721x over previous
import functools

import jax
import jax.numpy as jnp
from jax.experimental import pallas as pl

BS = 64
C = 128
S = 32
P = 32
M = 16
K = 2
EPS = 1e-5
BB = 8
F32 = jnp.float32


def _rows(x):
    return x.reshape(-1, x.shape[-1])


def _fiota(shape, dim):
    return jax.lax.broadcasted_iota(jnp.int32, shape, dim).astype(F32)


def _mm(a, w):
    return jnp.dot(a, w, preferred_element_type=F32)


def _sumsq(y, n_extra_rows=4):
    s1 = jnp.sum(y, axis=tuple(range(y.ndim - 1))).reshape(1, C)
    s2 = jnp.sum(y * y, axis=tuple(range(y.ndim - 1))).reshape(1, C)
    return jnp.concatenate([s1, s2] + [jnp.zeros((1, C), F32)] * n_extra_rows,
                           axis=0)


def _bn_coeffs(sum_row, sumsq_row, n, g, b):
    mean = sum_row * (1.0 / n)
    var = sumsq_row * (1.0 / n) - mean * mean
    scale = g * jax.lax.rsqrt(var + EPS)
    return scale, b - mean * scale


def _k1(x_ref, s_ref, wt_ref, tb_ref, sw_ref, sb_ref, t_ref, yud_ref,
        st_ref):
    i = pl.program_id(0)
    x = x_ref[...]
    z = jnp.zeros((BB, S, 1, C), F32)
    xp = jnp.concatenate([z, x, z], axis=2)
    acc = _mm(_rows(xp[:, :, 0:P, :]), wt_ref[0])
    acc += _mm(_rows(xp[:, :, 1:P + 1, :]), wt_ref[1])
    acc += _mm(_rows(xp[:, :, 2:P + 2, :]), wt_ref[2])
    t = acc.reshape(BB, S, P, C) + tb_ref[...]
    t_ref[...] = t

    sT = s_ref[...]
    yd = _mm(_rows(x), sw_ref[0:C, :]).reshape(BB, S, P, C)
    ys = _mm(_rows(sT), sw_ref[C:2 * C, :]).reshape(BB, S, 1, C)
    y = yd + ys + sb_ref[...]
    yud_ref[...] = y

    st = jnp.concatenate([_sumsq(t, 0), _sumsq(y, 0)], axis=0)
    st = jnp.concatenate([st, jnp.zeros((4, C), F32)], axis=0)

    @pl.when(i == 0)
    def _():
        st_ref[...] = jnp.zeros((8, C), F32)

    st_ref[...] += st


def _k2(t_ref, s_ref, st1_ref, g1_ref, b1_ref, mw_ref, mb_ref,
        yus_ref, st2_ref):
    i = pl.program_id(0)
    n1 = BS * S * P
    sc, sh = _bn_coeffs(st1_ref[0:1, :], st1_ref[1:2, :], n1,
                        g1_ref[...], b1_ref[...])
    a = jax.nn.gelu(t_ref[...] * sc + sh)
    sfd = jnp.max(a, axis=2)
    y = _mm(_rows(s_ref[...]), mw_ref[0:C, :])
    y += _mm(_rows(sfd), mw_ref[C:2 * C, :])
    y = y.reshape(BB, S, C) + mb_ref[...]
    yus_ref[...] = y

    @pl.when(i == 0)
    def _():
        st2_ref[...] = jnp.zeros((8, C), F32)

    st2_ref[...] += _sumsq(y, 6)


def _k3(c_ref, fps_ref, nb_ref, cs_ref):
    coor = c_ref[...]
    n2 = jnp.sum(coor * coor, axis=-1)
    cb = coor.astype(jnp.bfloat16)
    ip = jnp.concatenate(
        [jax.lax.dot_general(cb[b], cb[b], (((1,), (1,)), ((), ())),
                             preferred_element_type=F32)[None]
         for b in range(BS)], axis=0)
    dk = n2[:, :, None] + n2[:, None, :] - 2.0 * ip

    iota3 = _fiota((BS, S, S), 2)
    BIG = 1e30

    def first_min_idx(d):
        mn = jnp.min(d, axis=-1, keepdims=True)
        return jnp.min(jnp.where(d <= mn, iota3, BIG), axis=-1)

    i1 = first_min_idx(dk)
    dk2 = jnp.where(iota3 == i1[:, :, None], BIG, dk)
    i2 = first_min_idx(dk2)
    knn = jnp.concatenate([i1[:, :, None], i2[:, :, None]], axis=-1)

    iota2 = _fiota((BS, S), 1)
    dist = jnp.full((BS, S), 1e10, F32)
    far = jnp.zeros((BS, 1), F32)
    ohs = []
    for _ in range(M):
        oh = (iota2 == far).astype(F32)
        ohs.append(oh[:, None, :])
        centroid = jnp.sum(oh[:, :, None] * coor, axis=1)
        d = jnp.sum((coor - centroid[:, None, :]) ** 2, axis=-1)
        dist = jnp.minimum(dist, d)
        mx = jnp.max(dist, axis=-1, keepdims=True)
        far = jnp.min(jnp.where(dist >= mx, iota2, BIG), axis=-1,
                      keepdims=True)
    fps_oh = jnp.concatenate(ohs, axis=1)
    fps_ref[...] = fps_oh

    knn_fps = jnp.sum(fps_oh[:, :, :, None] * knn[:, None, :, :], axis=2)
    iota4 = _fiota((BS, M, K, S), 3)
    nb_ref[...] = (iota4 == knn_fps[:, :, :, None]).astype(F32)

    cs_ref[...] = jnp.sum(fps_oh[:, :, :, None] * coor[:, None, :, :],
                          axis=2)


def _k4(yus_ref, yud_ref, st2_ref, st13_ref, g2_ref, b2_ref, g3_ref, b3_ref,
        fps_ref, nb_ref, spw_ref, spb_ref, dnw_ref, dnb_ref,
        sp_ref, dn_ref, st45_ref):
    i = pl.program_id(0)
    sc2, sh2 = _bn_coeffs(st2_ref[0:1, :], st2_ref[1:2, :], BS * S,
                          g2_ref[...], b2_ref[...])
    us = jax.nn.gelu(yus_ref[...] * sc2 + sh2)
    fps_oh = fps_ref[...]
    nb_oh = nb_ref[...]

    cen = []
    nbr = []
    for b in range(BB):
        cen.append(_mm(fps_oh[b], us[b])[None])
        nbr.append(_mm(nb_oh[b].reshape(M * K, S), us[b])[None])
    cen = jnp.concatenate(cen, axis=0)
    nbr = jnp.concatenate(nbr, axis=0).reshape(BB, M, K, C)
    diff = nbr - cen[:, :, None, :]
    sp = _mm(_rows(diff), spw_ref[0:C, :]).reshape(BB, M, K, C)
    sp += _mm(_rows(cen), spw_ref[C:2 * C, :]).reshape(BB, M, 1, C)
    sp += spb_ref[...]
    sp_ref[...] = sp

    sc3, sh3 = _bn_coeffs(st13_ref[2:3, :], st13_ref[3:4, :], BS * S * P,
                          g3_ref[...], b3_ref[...])
    ud = jax.nn.gelu(yud_ref[...] * sc3 + sh3)
    udf = ud.reshape(BB, S, P * C)
    dc = []
    dnb = []
    for b in range(BB):
        dc.append(_mm(fps_oh[b], udf[b])[None])
        dnb.append(_mm(nb_oh[b].reshape(M * K, S), udf[b])[None])
    dc = jnp.concatenate(dc, axis=0)
    dnb = jnp.concatenate(dnb, axis=0).reshape(BB, M, K, P * C)
    ddiff = (dnb - dc[:, :, None, :]).reshape(BB, M, K, P // 2, 2, C)
    de = ddiff[:, :, :, :, 0, :]
    do = ddiff[:, :, :, :, 1, :]
    A = _mm(_rows(de), dnw_ref[0:C, :]) + _mm(_rows(do), dnw_ref[C:2 * C, :])
    A = A.reshape(BB, M, K, P // 2, C)
    dcr = dc.reshape(BB, M, P // 2, 2, C)
    Bc = _mm(_rows(dcr[:, :, :, 0, :]), dnw_ref[0:C, :])
    Bc += _mm(_rows(dcr[:, :, :, 1, :]), dnw_ref[C:2 * C, :])
    Bc = Bc.reshape(BB, M, 1, P // 2, C)
    dn = jnp.concatenate([A, jnp.broadcast_to(Bc, (BB, M, K, P // 2, C))],
                         axis=3) + dnb_ref[...]
    dn_ref[...] = dn

    st = jnp.concatenate([_sumsq(sp, 0), _sumsq(dn, 0)], axis=0)
    st = jnp.concatenate([st, jnp.zeros((4, C), F32)], axis=0)

    @pl.when(i == 0)
    def _():
        st45_ref[...] = jnp.zeros((8, C), F32)

    st45_ref[...] += st


def _k5a(sp_ref, dn_ref, st45_ref, g4_ref, b4_ref, g5_ref, b5_ref,
         dw_ref, db_ref, spo_ref, y_ref, st6_ref):
    i = pl.program_id(0)
    sc4, sh4 = _bn_coeffs(st45_ref[0:1, :], st45_ref[1:2, :], BS * M * K,
                          g4_ref[...], b4_ref[...])
    sp = jax.nn.gelu(sp_ref[...] * sc4 + sh4)
    spo_ref[...] = jnp.max(sp, axis=2)

    sc5, sh5 = _bn_coeffs(st45_ref[2:3, :], st45_ref[3:4, :], BS * M * K * P,
                          g5_ref[...], b5_ref[...])
    h = jax.nn.gelu(dn_ref[...] * sc5 + sh5)
    hm = jnp.max(h, axis=2)
    hr = hm.reshape(BB, M, P // 2, 2, C)
    he = hr[:, :, :, 0, :]
    ho = hr[:, :, :, 1, :]
    hop = jnp.concatenate([jnp.zeros((BB, M, 1, C), F32),
                           ho[:, :, :P // 2 - 1, :]], axis=2)
    y = _mm(_rows(hop), dw_ref[0]) + _mm(_rows(he), dw_ref[1])
    y += _mm(_rows(ho), dw_ref[2])
    y = y.reshape(BB, M, P // 2, C) + db_ref[...]
    y_ref[...] = y

    @pl.when(i == 0)
    def _():
        st6_ref[...] = jnp.zeros((8, C), F32)

    st6_ref[...] += _sumsq(y, 6)


def _k5b(y_ref, st6_ref, g6_ref, b6_ref, do_ref):
    n6 = BS * M * (P // 2)
    sc6, sh6 = _bn_coeffs(st6_ref[0:1, :], st6_ref[1:2, :], n6,
                          g6_ref[...], b6_ref[...])
    do_ref[...] = jax.nn.gelu(y_ref[...] * sc6 + sh6)


def _full(shape):
    nd = len(shape)
    return pl.BlockSpec(shape, lambda *_: (0,) * nd)


def _batched(shape):
    nd = len(shape)
    return pl.BlockSpec((BB,) + shape[1:],
                        lambda i: (i,) + (0,) * (nd - 1))


def kernel(sparse_fea, dense_fea, stk_coor, dts_tconv_w, dts_tconv_b,
           dts_tbn_g, dts_tbn_b, dts_mlp_w, dts_mlp_b, dts_bn_g, dts_bn_b,
           std_mlp_w, std_mlp_b, std_bn_g, std_bn_b, sp_conv_w, sp_conv_b,
           sp_bn_g, sp_bn_b, dn_conv_w, dn_conv_b, dn_bn_g, dn_bn_b,
           dn_down_w, dn_down_b, dn_dbn_g, dn_dbn_b):
    f32 = F32
    xT = jnp.transpose(dense_fea, (0, 2, 3, 1))
    sT = jnp.transpose(sparse_fea, (0, 2, 1))
    wt = jnp.transpose(dts_tconv_w[:, :, 0, :], (2, 1, 0))
    dwt = jnp.transpose(dn_down_w[:, :, 0, :], (2, 1, 0))
    row = lambda v: v.reshape(1, C)

    nsteps = BS // BB
    t, yud, st13 = pl.pallas_call(
        _k1,
        grid=(nsteps,),
        in_specs=[_batched((BS, S, P, C)), _batched((BS, S, C)),
                  _full((3, C, C)), _full((1, C)), _full((2 * C, C)),
                  _full((1, C))],
        out_specs=[_batched((BS, S, P, C)), _batched((BS, S, P, C)),
                   _full((8, C))],
        out_shape=[jax.ShapeDtypeStruct((BS, S, P, C), f32),
                   jax.ShapeDtypeStruct((BS, S, P, C), f32),
                   jax.ShapeDtypeStruct((8, C), f32)],
    )(xT, sT, wt, row(dts_tconv_b), std_mlp_w, row(std_mlp_b))

    yus, st2 = pl.pallas_call(
        _k2,
        grid=(nsteps,),
        in_specs=[_batched((BS, S, P, C)), _batched((BS, S, C)),
                  _full((8, C)), _full((1, C)), _full((1, C)),
                  _full((2 * C, C)), _full((1, C))],
        out_specs=[_batched((BS, S, C)), _full((8, C))],
        out_shape=[jax.ShapeDtypeStruct((BS, S, C), f32),
                   jax.ShapeDtypeStruct((8, C), f32)],
    )(t, sT, st13, row(dts_tbn_g), row(dts_tbn_b), dts_mlp_w,
      row(dts_mlp_b))

    fps_oh, nb_oh, coor_s = pl.pallas_call(
        _k3,
        in_specs=[_full((BS, S, 3))],
        out_specs=[_full((BS, M, S)), _full((BS, M, K, S)),
                   _full((BS, M, 3))],
        out_shape=[jax.ShapeDtypeStruct((BS, M, S), f32),
                   jax.ShapeDtypeStruct((BS, M, K, S), f32),
                   jax.ShapeDtypeStruct((BS, M, 3), f32)],
    )(stk_coor)

    sp_pre, dn_pre, st45 = pl.pallas_call(
        _k4,
        grid=(nsteps,),
        in_specs=[_batched((BS, S, C)), _batched((BS, S, P, C)),
                  _full((8, C)), _full((8, C)), _full((1, C)),
                  _full((1, C)), _full((1, C)), _full((1, C)),
                  _batched((BS, M, S)), _batched((BS, M, K, S)),
                  _full((2 * C, C)), _full((1, C)), _full((2 * C, C)),
                  _full((1, C))],
        out_specs=[_batched((BS, M, K, C)), _batched((BS, M, K, P, C)),
                   _full((8, C))],
        out_shape=[jax.ShapeDtypeStruct((BS, M, K, C), f32),
                   jax.ShapeDtypeStruct((BS, M, K, P, C), f32),
                   jax.ShapeDtypeStruct((8, C), f32)],
    )(yus, yud, st2, st13, row(dts_bn_g), row(dts_bn_b), row(std_bn_g),
      row(std_bn_b), fps_oh, nb_oh, sp_conv_w, row(sp_conv_b), dn_conv_w,
      row(dn_conv_b))

    sp_out, y6, st6 = pl.pallas_call(
        _k5a,
        grid=(nsteps,),
        in_specs=[_batched((BS, M, K, C)), _batched((BS, M, K, P, C)),
                  _full((8, C)), _full((1, C)), _full((1, C)),
                  _full((1, C)), _full((1, C)), _full((3, C, C)),
                  _full((1, C))],
        out_specs=[_batched((BS, M, C)), _batched((BS, M, P // 2, C)),
                   _full((8, C))],
        out_shape=[jax.ShapeDtypeStruct((BS, M, C), f32),
                   jax.ShapeDtypeStruct((BS, M, P // 2, C), f32),
                   jax.ShapeDtypeStruct((8, C), f32)],
    )(sp_pre, dn_pre, st45, row(sp_bn_g), row(sp_bn_b), row(dn_bn_g),
      row(dn_bn_b), dwt, row(dn_down_b))

    d_out = pl.pallas_call(
        _k5b,
        grid=(nsteps,),
        in_specs=[_batched((BS, M, P // 2, C)), _full((8, C)),
                  _full((1, C)), _full((1, C))],
        out_specs=_batched((BS, M, P // 2, C)),
        out_shape=jax.ShapeDtypeStruct((BS, M, P // 2, C), f32),
    )(y6, st6, row(dn_dbn_g), row(dn_dbn_b))

    sp_final = jnp.transpose(sp_out, (0, 2, 1))
    d_final = jnp.transpose(d_out, (0, 3, 1, 2))
    return (sp_final, d_final, coor_s)
